# Initial kernel scaffold; baseline (speedup 1.0000x reference)
import jax
import jax.numpy as jnp
from jax.experimental import pallas as pl


def kernel(x, edge_index, node_batch, W_in, b_in, W_lv, b_lv, ln_g, ln_b, p1_W, p1_b, bn_g, bn_b, p2_W, p2_b):
    raise NotImplementedError("write your pallas kernel here")


# speedup vs baseline: 6.6385x; 6.6385x over previous
import functools

import jax
import jax.numpy as jnp
from jax import lax
from jax.experimental import pallas as pl
from jax.experimental.pallas import tpu as pltpu
from jax.experimental.pallas import tpu_sc as plsc

_NC = 2
_NS = 16
_NW = _NC * _NS
_CH = 128

_PREC = lax.Precision.HIGHEST


def _seg_sum_sc(N_pad, H, n_chunks):
  mesh = plsc.VectorSubcoreMesh(core_axis_name="c", subcore_axis_name="s",
                                num_cores=_NC, num_subcores=_NS)
  rows_per_tile = N_pad // _NS

  @functools.partial(
      pl.kernel,
      out_type=jax.ShapeDtypeStruct((_NC, N_pad, H), jnp.float32),
      mesh=mesh,
      scratch_types=[
          pltpu.VMEM((n_chunks, _CH), jnp.int32),
          pltpu.VMEM((n_chunks, _CH), jnp.int32),
          pltpu.VMEM((_CH, H), jnp.float32),
          pltpu.VMEM_SHARED((N_pad, H), jnp.float32),
          pltpu.SemaphoreType.DMA,
      ],
      compiler_params=pltpu.CompilerParams(use_tc_tiling_on_sc=False),
  )
  def seg(h_hbm, src_hbm, dst_hbm, zeros_hbm, out_hbm,
          src_v, dst_v, rows_v, agg_sh, sem):
    cid = lax.axis_index("c")
    sid = lax.axis_index("s")
    wid = sid * _NC + cid
    r0 = sid * rows_per_tile
    pltpu.sync_copy(zeros_hbm.at[pl.ds(r0, rows_per_tile)],
                    agg_sh.at[pl.ds(r0, rows_per_tile)])
    pltpu.sync_copy(src_hbm.at[wid], src_v)
    pltpu.sync_copy(dst_hbm.at[wid], dst_v)
    plsc.subcore_barrier()

    @pl.loop(0, n_chunks)
    def _chunk(i):
      pltpu.async_copy(h_hbm.at[src_v.at[i]], rows_v, sem).wait()
      pltpu.sync_copy(rows_v, agg_sh.at[dst_v.at[i]], add=True)

    plsc.subcore_barrier()
    pltpu.sync_copy(agg_sh.at[pl.ds(r0, rows_per_tile)],
                    out_hbm.at[cid, pl.ds(r0, rows_per_tile)])

  return seg


def _embed_tc(x_ref, w_ref, b_ref, o_ref):
  o_ref[...] = jnp.maximum(
      jnp.dot(x_ref[...], w_ref[...], preferred_element_type=jnp.float32)
      + b_ref[...], 0.0)


def _make_level_tc(N, H):
  def body(h_ref, parts_ref, w_ref, b_ref, g_ref, bb_ref, o_ref):
    agg = parts_ref[0, :N, :] + parts_ref[1, :N, :]
    comb = jnp.dot(jnp.concatenate([h_ref[...], agg], axis=-1), w_ref[...],
                   preferred_element_type=jnp.float32) + b_ref[...]
    mu = jnp.mean(comb, axis=-1, keepdims=True)
    var = jnp.mean((comb - mu) ** 2, axis=-1, keepdims=True)
    y = (comb - mu) / jnp.sqrt(var + 1e-5) * g_ref[...] + bb_ref[...]
    o_ref[...] = jnp.maximum(y, 0.0)
  return body


def _make_readout_tc(B):
  def body(nb_ref, h_ref, p1w_ref, p1b_ref, bng_ref, bnb_ref,
           p2w_ref, p2b_ref, o_ref):
    nb = nb_ref[...]
    seg_ids = lax.broadcasted_iota(jnp.int32, (B, nb.shape[1]), 0)
    m = (nb == seg_ids).astype(jnp.float32)
    gr = jnp.dot(m, h_ref[...], preferred_element_type=jnp.float32,
                 precision=_PREC)
    z = jnp.dot(gr, p1w_ref[...],
                preferred_element_type=jnp.float32) + p1b_ref[...]
    z = jnp.maximum(z * bng_ref[...] + bnb_ref[...], 0.0)
    o_ref[...] = jnp.dot(z, p2w_ref[...],
                         preferred_element_type=jnp.float32) + p2b_ref[...]
  return body


def kernel(x, edge_index, node_batch, W_in, b_in, W_lv, b_lv, ln_g, ln_b,
           p1_W, p1_b, bn_g, bn_b, p2_W, p2_b):
  N, D = x.shape
  H = W_in.shape[1]
  E = edge_index.shape[1]
  HEIGHT = W_lv.shape[0]
  B = 128

  n_chunks = -(-E // (_NW * _CH))
  E_pad = n_chunks * _NW * _CH
  N_pad = ((N + 1 + _NS * 8 - 1) // (_NS * 8)) * (_NS * 8)

  src = edge_index[0]
  dst = edge_index[1]
  pad = E_pad - E
  if pad:
    src = jnp.concatenate([src, jnp.zeros((pad,), jnp.int32)])
    dst = jnp.concatenate([dst, jnp.full((pad,), N_pad - 1, jnp.int32)])
  src_r = src.reshape(_NW, n_chunks, _CH)
  dst_r = dst.reshape(_NW, n_chunks, _CH)
  zeros = jnp.zeros((N_pad, H), jnp.float32)

  seg = _seg_sum_sc(N_pad, H, n_chunks)
  level = _make_level_tc(N, H)

  h = pl.pallas_call(
      _embed_tc,
      out_shape=jax.ShapeDtypeStruct((N, H), jnp.float32),
  )(x, W_in, b_in.reshape(1, H))

  for l in range(HEIGHT):
    parts = seg(h, src_r, dst_r, zeros)
    h = pl.pallas_call(
        level,
        out_shape=jax.ShapeDtypeStruct((N, H), jnp.float32),
    )(h, parts, W_lv[l], b_lv[l].reshape(1, H),
      ln_g[l].reshape(1, H), ln_b[l].reshape(1, H))

  p2w_pad = jnp.pad(p2_W, ((0, 0), (0, 127)))
  p2b_pad = jnp.broadcast_to(p2_b.reshape(1, 1), (1, 128))
  out = pl.pallas_call(
      _make_readout_tc(B),
      out_shape=jax.ShapeDtypeStruct((B, 128), jnp.float32),
  )(node_batch.reshape(1, N), h, p1_W, p1_b.reshape(1, H),
    bn_g.reshape(1, H), bn_b.reshape(1, H), p2w_pad, p2b_pad)
  return out[:, :1]
